# baseline (speedup 1.0000x reference)
import functools

import jax
import jax.numpy as jnp
from jax import lax
from jax.experimental import pallas as pl
from jax.experimental.pallas import tpu as pltpu
from jax.experimental.pallas import tpu_sc as plsc

_EPS = 0.1
_ALPHA = 0.2
_CB = 3328
_RB = 128
_SLAB = 16

_NUM_CORES = 2
_NUM_SUBCORES = 16
_NUM_WORKERS = _NUM_CORES * _NUM_SUBCORES


def _slab_start(p3, C):
    return jnp.minimum(p3 >> 3, (C - _SLAB) >> 3) * 8


def _sc_slabs_body(C, BPW, x_hbm, pid_hbm, win_hbm, pid_v, win_v, sem):
    wid = lax.axis_index("s") * _NUM_CORES + lax.axis_index("c")
    base = wid * BPW
    pltpu.sync_copy(pid_hbm.at[pl.ds(base, BPW)], pid_v)
    copies = []
    for r in range(BPW):
        chunk = pid_v[pl.ds((r // 16) * 16, 16)]
        p3 = chunk[r % 16] * 3
        start = _slab_start(p3, C)
        cg = ((base + r) >> 7) * 128
        copies.append(
            pltpu.async_copy(x_hbm.at[pl.ds(start, _SLAB), pl.ds(cg, 128)],
                             win_v.at[r], sem))
    for cp in copies:
        cp.wait()
    pltpu.sync_copy(win_v, win_hbm.at[pl.ds(base, BPW)])


def _sc_slabs(xt, pids, B, C):
    BPW = B // _NUM_WORKERS
    mesh = plsc.VectorSubcoreMesh(core_axis_name="c", subcore_axis_name="s")
    f = functools.partial(
        pl.kernel,
        mesh=mesh,
        out_type=jax.ShapeDtypeStruct((B, _SLAB, 128), jnp.float32),
        scratch_types=[
            pltpu.VMEM((BPW,), jnp.int32),
            pltpu.VMEM((BPW, _SLAB, 128), jnp.float32),
            pltpu.SemaphoreType.DMA,
        ],
    )(functools.partial(_sc_slabs_body, C, BPW))
    return f(xt, pids)


def _stream_body(C, x_ref, m_out, z_out, s_out, tail_out, macc, zacc, sacc):
    i = pl.program_id(0)
    n = pl.num_programs(0)
    x = x_ref[...]
    Cb, B = x.shape

    @pl.when(i == 0)
    def _():
        macc[...] = jnp.full_like(macc, -jnp.inf)
        zacc[...] = jnp.zeros_like(zacc)
        sacc[...] = jnp.zeros_like(sacc)

    def update(xv, xm):
        bm = jnp.max(xm, axis=0, keepdims=True)
        m_new = jnp.maximum(macc[...], bm)
        scale = jnp.exp(macc[...] - m_new)
        zacc[...] = zacc[...] * scale + jnp.sum(
            jnp.exp(xm - m_new), axis=0, keepdims=True)
        sacc[...] += jnp.sum(xv, axis=0, keepdims=True)
        macc[...] = m_new

    @pl.when(i < n - 1)
    def _():
        update(x, x)

    @pl.when(i == n - 1)
    def _():
        row = lax.broadcasted_iota(jnp.int32, x.shape, 0) + i * Cb
        valid = row < C
        xv = jnp.where(valid, x, jnp.zeros_like(x))
        xm = jnp.where(valid, x, jnp.full_like(x, -jnp.inf))
        update(xv, xm)
        m_out[...] = macc[...]
        z_out[...] = zacc[...]
        s_out[...] = sacc[...]
        lo = C - 8 - (n - 1) * Cb
        tail_out[...] = lax.slice(x, (lo, 0), (lo + 8, B))


def _stream(xt, B, C):
    n = (C + _CB - 1) // _CB
    return pl.pallas_call(
        functools.partial(_stream_body, C),
        grid=(n,),
        in_specs=[pl.BlockSpec((_CB, B), lambda i: (i, 0))],
        out_specs=[
            pl.BlockSpec((1, B), lambda i: (0, 0)),
            pl.BlockSpec((1, B), lambda i: (0, 0)),
            pl.BlockSpec((1, B), lambda i: (0, 0)),
            pl.BlockSpec((8, B), lambda i: (0, 0)),
        ],
        out_shape=[
            jax.ShapeDtypeStruct((1, B), jnp.float32),
            jax.ShapeDtypeStruct((1, B), jnp.float32),
            jax.ShapeDtypeStruct((1, B), jnp.float32),
            jax.ShapeDtypeStruct((8, B), jnp.float32),
        ],
        scratch_shapes=[
            pltpu.VMEM((1, B), jnp.float32),
            pltpu.VMEM((1, B), jnp.float32),
            pltpu.VMEM((1, B), jnp.float32),
        ],
    )(xt)


def _combine_body(C, m_ref, z_ref, s_ref, tail_ref, win_ref, pid_ref, vid_ref,
                  out_ref):
    i = pl.program_id(0)
    n = pl.num_programs(0)
    m = m_ref[...]
    z = z_ref[...]
    s = s_ref[...]
    win = win_ref[...]
    tail = tail_ref[...]
    p3 = pid_ref[...] * 3
    vid = vid_ref[...]

    d0 = lax.broadcasted_iota(jnp.int32, win.shape, 0)
    d2 = lax.broadcasted_iota(jnp.int32, win.shape, 2)
    wcol = jnp.sum(jnp.where(d2 == d0, win, jnp.zeros_like(win)), axis=2)

    start = _slab_start(p3, C)
    rowg = lax.broadcasted_iota(jnp.int32, wcol.shape, 1) + start
    rowt = lax.broadcasted_iota(jnp.int32, tail.shape, 1) + (C - 8)
    A = ((C - _SLAB) >> 3 << 3) + _SLAB
    zs = jnp.zeros_like(wcol)
    zt = jnp.zeros_like(tail)

    def pick(q):
        gw = jnp.sum(jnp.where(rowg == q, wcol, zs), axis=1, keepdims=True)
        gt_ = jnp.sum(jnp.where((rowt == q) & (rowt >= A), tail, zt),
                      axis=1, keepdims=True)
        return gw + gt_

    g0 = pick(p3)
    g1 = pick(p3 + 1)
    g2 = pick(p3 + 2)
    gv = pick(p3 + vid)

    c = m + jnp.log(z)
    ep1 = jnp.exp(g0 - c) + jnp.exp(g1 - c) + jnp.exp(g2 - c)
    ep2 = jnp.exp(gv - c)
    L = s - C * c
    L3 = (g0 + g1 + g2) - 3.0 * c
    lpt = gv - c
    e1 = _ALPHA * (1.0 - ep1)
    e2 = _ALPHA * (1.0 - ep2)
    S = (e1 / (C - 3)) * (L - L3) + 0.5 * e2 * (L3 - lpt) + (1.0 - e1 - e2) * lpt
    contrib = (1.0 - _EPS) * S + (_EPS / C) * L
    bs = jnp.sum(contrib, axis=0, keepdims=True)

    @pl.when(i == 0)
    def _():
        out_ref[...] = jnp.zeros_like(out_ref)

    out_ref[...] += bs

    @pl.when(i == n - 1)
    def _():
        B_total = n * m.shape[0]
        out_ref[...] = out_ref[...] * (-1.0 / B_total)


def _combine(m, z, s, tail_t, win, pids2, vids2, B, C):
    n = B // _RB
    return pl.pallas_call(
        functools.partial(_combine_body, C),
        grid=(n,),
        in_specs=[
            pl.BlockSpec((_RB, 1), lambda i: (i, 0)),
            pl.BlockSpec((_RB, 1), lambda i: (i, 0)),
            pl.BlockSpec((_RB, 1), lambda i: (i, 0)),
            pl.BlockSpec((_RB, 8), lambda i: (i, 0)),
            pl.BlockSpec((_RB, _SLAB, 128), lambda i: (i, 0, 0)),
            pl.BlockSpec((_RB, 1), lambda i: (i, 0)),
            pl.BlockSpec((_RB, 1), lambda i: (i, 0)),
        ],
        out_specs=pl.BlockSpec((1, 1), lambda i: (0, 0)),
        out_shape=jax.ShapeDtypeStruct((1, 1), jnp.float32),
    )(m, z, s, tail_t, win, pids2, vids2)


@jax.jit
def kernel(inputs, pids, vids):
    B, C = inputs.shape
    xt = inputs.T
    pids32 = pids.astype(jnp.int32)
    win = _sc_slabs(xt, pids32, B, C)
    m, z, s, tail = _stream(xt, B, C)
    out = _combine(m.reshape(B, 1), z.reshape(B, 1), s.reshape(B, 1),
                   tail.T, win, pids32.reshape(B, 1),
                   vids.reshape(B, 1).astype(jnp.int32), B, C)
    return out[0, 0]


# speedup vs baseline: 1.0185x; 1.0185x over previous
import functools

import jax
import jax.numpy as jnp
from jax import lax
from jax.experimental import pallas as pl
from jax.experimental.pallas import tpu as pltpu
from jax.experimental.pallas import tpu_sc as plsc

_EPS = 0.1
_ALPHA = 0.2
_CB = 3072
_RB = 256
_SLAB = 16

_NUM_CORES = 2
_NUM_SUBCORES = 16
_NUM_WORKERS = _NUM_CORES * _NUM_SUBCORES


def _slab_start(p3, C):
    return jnp.minimum(p3 >> 3, (C - _SLAB) >> 3) * 8


def _sc_slabs_body(C, BPW, x_hbm, pid_hbm, win_hbm, pid_v, win_v, sem):
    wid = lax.axis_index("s") * _NUM_CORES + lax.axis_index("c")
    base = wid * BPW
    pltpu.sync_copy(pid_hbm.at[pl.ds(base, BPW)], pid_v)
    copies = []
    for r in range(BPW):
        chunk = pid_v[pl.ds((r // 16) * 16, 16)]
        p3 = chunk[r % 16] * 3
        start = _slab_start(p3, C)
        cg = ((base + r) >> 7) * 128
        copies.append(
            pltpu.async_copy(x_hbm.at[pl.ds(start, _SLAB), pl.ds(cg, 128)],
                             win_v.at[r], sem))
    for cp in copies:
        cp.wait()
    pltpu.sync_copy(win_v, win_hbm.at[pl.ds(base, BPW)])


def _sc_slabs(xt, pids, B, C):
    BPW = B // _NUM_WORKERS
    mesh = plsc.VectorSubcoreMesh(core_axis_name="c", subcore_axis_name="s")
    f = functools.partial(
        pl.kernel,
        mesh=mesh,
        out_type=jax.ShapeDtypeStruct((B, _SLAB, 128), jnp.float32),
        scratch_types=[
            pltpu.VMEM((BPW,), jnp.int32),
            pltpu.VMEM((BPW, _SLAB, 128), jnp.float32),
            pltpu.SemaphoreType.DMA,
        ],
    )(functools.partial(_sc_slabs_body, C, BPW))
    return f(xt, pids)


def _stream_body(C, x_ref, m_out, z_out, s_out, tail_out, macc, zacc, sacc):
    i = pl.program_id(0)
    n = pl.num_programs(0)
    x = x_ref[...]
    Cb, B = x.shape

    @pl.when(i == 0)
    def _():
        macc[...] = jnp.full_like(macc, -jnp.inf)
        zacc[...] = jnp.zeros_like(zacc)
        sacc[...] = jnp.zeros_like(sacc)

    def update(xv, xm):
        bm = jnp.max(xm, axis=0, keepdims=True)
        m_new = jnp.maximum(macc[...], bm)
        scale = jnp.exp(macc[...] - m_new)
        zacc[...] = zacc[...] * scale + jnp.sum(
            jnp.exp(xm - m_new), axis=0, keepdims=True)
        sacc[...] += jnp.sum(xv, axis=0, keepdims=True)
        macc[...] = m_new

    @pl.when(i < n - 1)
    def _():
        update(x, x)

    @pl.when(i == n - 1)
    def _():
        row = lax.broadcasted_iota(jnp.int32, x.shape, 0) + i * Cb
        valid = row < C
        xv = jnp.where(valid, x, jnp.zeros_like(x))
        xm = jnp.where(valid, x, jnp.full_like(x, -jnp.inf))
        update(xv, xm)
        m_out[...] = macc[...]
        z_out[...] = zacc[...]
        s_out[...] = sacc[...]
        lo = C - 8 - (n - 1) * Cb
        tail_out[...] = lax.slice(x, (lo, 0), (lo + 8, B))


def _stream(xt, B, C):
    n = (C + _CB - 1) // _CB
    return pl.pallas_call(
        functools.partial(_stream_body, C),
        grid=(n,),
        in_specs=[pl.BlockSpec((_CB, B), lambda i: (i, 0))],
        out_specs=[
            pl.BlockSpec((1, B), lambda i: (0, 0)),
            pl.BlockSpec((1, B), lambda i: (0, 0)),
            pl.BlockSpec((1, B), lambda i: (0, 0)),
            pl.BlockSpec((8, B), lambda i: (0, 0)),
        ],
        out_shape=[
            jax.ShapeDtypeStruct((1, B), jnp.float32),
            jax.ShapeDtypeStruct((1, B), jnp.float32),
            jax.ShapeDtypeStruct((1, B), jnp.float32),
            jax.ShapeDtypeStruct((8, B), jnp.float32),
        ],
        scratch_shapes=[
            pltpu.VMEM((1, B), jnp.float32),
            pltpu.VMEM((1, B), jnp.float32),
            pltpu.VMEM((1, B), jnp.float32),
        ],
    )(xt)


def _combine_body(C, m_ref, z_ref, s_ref, tail_ref, win_ref, pid_ref, vid_ref,
                  out_ref):
    i = pl.program_id(0)
    n = pl.num_programs(0)
    m = m_ref[...]
    z = z_ref[...]
    s = s_ref[...]
    win = win_ref[...]
    tail = tail_ref[...]
    p3 = pid_ref[...] * 3
    vid = vid_ref[...]

    d0 = lax.broadcasted_iota(jnp.int32, win.shape, 0)
    d2 = lax.broadcasted_iota(jnp.int32, win.shape, 2)
    wcol = jnp.sum(jnp.where(d2 == d0, win, jnp.zeros_like(win)), axis=2)

    start = _slab_start(p3, C)
    rowg = lax.broadcasted_iota(jnp.int32, wcol.shape, 1) + start
    rowt = lax.broadcasted_iota(jnp.int32, tail.shape, 1) + (C - 8)
    A = ((C - _SLAB) >> 3 << 3) + _SLAB
    zs = jnp.zeros_like(wcol)
    zt = jnp.zeros_like(tail)

    def pick(q):
        gw = jnp.sum(jnp.where(rowg == q, wcol, zs), axis=1, keepdims=True)
        gt_ = jnp.sum(jnp.where((rowt == q) & (rowt >= A), tail, zt),
                      axis=1, keepdims=True)
        return gw + gt_

    g0 = pick(p3)
    g1 = pick(p3 + 1)
    g2 = pick(p3 + 2)
    gv = pick(p3 + vid)

    c = m + jnp.log(z)
    ep1 = jnp.exp(g0 - c) + jnp.exp(g1 - c) + jnp.exp(g2 - c)
    ep2 = jnp.exp(gv - c)
    L = s - C * c
    L3 = (g0 + g1 + g2) - 3.0 * c
    lpt = gv - c
    e1 = _ALPHA * (1.0 - ep1)
    e2 = _ALPHA * (1.0 - ep2)
    S = (e1 / (C - 3)) * (L - L3) + 0.5 * e2 * (L3 - lpt) + (1.0 - e1 - e2) * lpt
    contrib = (1.0 - _EPS) * S + (_EPS / C) * L
    bs = jnp.sum(contrib, axis=0, keepdims=True)

    @pl.when(i == 0)
    def _():
        out_ref[...] = jnp.zeros_like(out_ref)

    out_ref[...] += bs

    @pl.when(i == n - 1)
    def _():
        B_total = n * m.shape[0]
        out_ref[...] = out_ref[...] * (-1.0 / B_total)


def _combine(m, z, s, tail_t, win, pids2, vids2, B, C):
    n = B // _RB
    return pl.pallas_call(
        functools.partial(_combine_body, C),
        grid=(n,),
        in_specs=[
            pl.BlockSpec((_RB, 1), lambda i: (i, 0)),
            pl.BlockSpec((_RB, 1), lambda i: (i, 0)),
            pl.BlockSpec((_RB, 1), lambda i: (i, 0)),
            pl.BlockSpec((_RB, 8), lambda i: (i, 0)),
            pl.BlockSpec((_RB, _SLAB, 128), lambda i: (i, 0, 0)),
            pl.BlockSpec((_RB, 1), lambda i: (i, 0)),
            pl.BlockSpec((_RB, 1), lambda i: (i, 0)),
        ],
        out_specs=pl.BlockSpec((1, 1), lambda i: (0, 0)),
        out_shape=jax.ShapeDtypeStruct((1, 1), jnp.float32),
    )(m, z, s, tail_t, win, pids2, vids2)


@jax.jit
def kernel(inputs, pids, vids):
    B, C = inputs.shape
    xt = inputs.T
    pids32 = pids.astype(jnp.int32)
    win = _sc_slabs(xt, pids32, B, C)
    m, z, s, tail = _stream(xt, B, C)
    out = _combine(m.reshape(B, 1), z.reshape(B, 1), s.reshape(B, 1),
                   tail.T, win, pids32.reshape(B, 1),
                   vids.reshape(B, 1).astype(jnp.int32), B, C)
    return out[0, 0]
